# baseline (speedup 1.0000x reference)
import functools

import jax
import jax.numpy as jnp
from jax import lax
from jax.experimental import pallas as pl
from jax.experimental.pallas import tpu as pltpu
from jax.experimental.pallas import tpu_sc as plsc

N = 10000
D = 256
DH = 128
N_PAD = 10240
NSUB = 16
ROWS_PER_TILE = N_PAD // NSUB
K = 128
BR = 640
GRID = N_PAD // BR
EPS = 1e-5


def _sc_mesh():
    return plsc.VectorSubcoreMesh(core_axis_name="c", subcore_axis_name="s")


def _sc_degree(dst3):
    n_chunks = dst3.shape[1]

    @functools.partial(
        pl.kernel,
        out_type=jax.ShapeDtypeStruct((N_PAD,), jnp.float32),
        mesh=_sc_mesh(),
        scratch_types=[
            pltpu.VMEM((n_chunks, K), jnp.int32),
            pltpu.VMEM((K,), jnp.float32),
            pltpu.VMEM((ROWS_PER_TILE,), jnp.float32),
            pltpu.VMEM_SHARED((N_PAD,), jnp.float32),
        ],
    )
    def k(dst_hbm, deg_hbm, dst_v, ones_v, zrow_v, accum):
        c = lax.axis_index("c")
        s = lax.axis_index("s")

        @pl.when(c == 0)
        def _():
            row0 = s * ROWS_PER_TILE
            pltpu.sync_copy(dst_hbm.at[s], dst_v)
            for i in range(K // 16):
                ones_v[pl.ds(i * 16, 16)] = jnp.ones((16,), jnp.float32)
            for i in range(ROWS_PER_TILE // 16):
                zrow_v[pl.ds(i * 16, 16)] = jnp.zeros((16,), jnp.float32)
            pltpu.sync_copy(zrow_v, accum.at[pl.ds(row0, ROWS_PER_TILE)])
            plsc.subcore_barrier()

            def body(j, carry):
                pltpu.sync_copy(ones_v, accum.at[dst_v.at[j]], add=True)
                return carry

            lax.fori_loop(0, n_chunks, body, 0)
            plsc.subcore_barrier()
            pltpu.sync_copy(accum.at[pl.ds(row0, ROWS_PER_TILE)],
                            deg_hbm.at[pl.ds(row0, ROWS_PER_TILE)])

    return k(dst3)


def _sc_scatter(y0, y1, src3, dst3):
    n_chunks = src3.shape[1]
    ph0 = (n_chunks + 1) // 2
    phases = [(0, ph0), (ph0, n_chunks - ph0)]

    @functools.partial(
        pl.kernel,
        out_type=[jax.ShapeDtypeStruct((N_PAD, DH), jnp.float32),
                  jax.ShapeDtypeStruct((N_PAD, DH), jnp.float32)],
        mesh=_sc_mesh(),
        scratch_types=[
            pltpu.VMEM((ph0, K), jnp.int32),
            pltpu.VMEM((ph0, K), jnp.int32),
            pltpu.VMEM((K, DH), jnp.float32),
            pltpu.VMEM((K, DH), jnp.float32),
            pltpu.VMEM_SHARED((N_PAD, DH), jnp.float32),
            pltpu.SemaphoreType.DMA,
            pltpu.SemaphoreType.DMA,
            pltpu.SemaphoreType.DMA,
            pltpu.SemaphoreType.DMA,
        ],
    )
    def k(y0_hbm, y1_hbm, src_hbm, dst_hbm, s0_hbm, s1_hbm,
          src_v, dst_v, gbuf0, gbuf1, accum, sem0, sem1, ssem0, ssem1):
        c = lax.axis_index("c")
        s = lax.axis_index("s")
        row0 = s * ROWS_PER_TILE

        def half(y_hbm, s_out):
            pltpu.sync_copy(y_hbm.at[pl.ds(row0, ROWS_PER_TILE)],
                            accum.at[pl.ds(row0, ROWS_PER_TILE)])
            plsc.subcore_barrier()

            def gather_start(j, buf, sem):
                pltpu.make_async_copy(
                    y_hbm.at[src_v.at[j, pl.ds(0, K // 2)]],
                    buf.at[pl.ds(0, K // 2)], sem).start()
                pltpu.make_async_copy(
                    y_hbm.at[src_v.at[j, pl.ds(K // 2, K // 2)]],
                    buf.at[pl.ds(K // 2, K // 2)], sem).start()

            def gather_wait(j, buf, sem):
                pltpu.make_async_copy(
                    y_hbm.at[src_v.at[j, pl.ds(0, K // 2)]],
                    buf.at[pl.ds(0, K // 2)], sem).wait()
                pltpu.make_async_copy(
                    y_hbm.at[src_v.at[j, pl.ds(K // 2, K // 2)]],
                    buf.at[pl.ds(K // 2, K // 2)], sem).wait()

            def scat(j, buf):
                pltpu.sync_copy(buf, accum.at[dst_v.at[j]], add=True)

            for lo, ph in phases:
                pltpu.sync_copy(src_hbm.at[s, pl.ds(lo, ph)],
                                src_v.at[pl.ds(0, ph)])
                pltpu.sync_copy(dst_hbm.at[s, pl.ds(lo, ph)],
                                dst_v.at[pl.ds(0, ph)])
                gather_start(0, gbuf0, sem0)

                def body(t, carry):
                    j = 2 * t
                    gather_start(j + 1, gbuf1, sem1)
                    gather_wait(j, gbuf0, sem0)
                    scat(j, gbuf0)

                    @pl.when(j + 2 < ph)
                    def _():
                        gather_start(j + 2, gbuf0, sem0)

                    gather_wait(j + 1, gbuf1, sem1)
                    scat(j + 1, gbuf1)
                    return carry

                lax.fori_loop(0, ph // 2, body, 0)
                if ph % 2:
                    gather_wait(ph - 1, gbuf0, sem0)
                    scat(ph - 1, gbuf0)

            plsc.subcore_barrier()
            pltpu.sync_copy(accum.at[pl.ds(row0, ROWS_PER_TILE)],
                            s_out.at[pl.ds(row0, ROWS_PER_TILE)])

        @pl.when(c == 0)
        def _():
            half(y0_hbm, s0_hbm)

        @pl.when(c == 1)
        def _():
            half(y1_hbm, s1_hbm)

    return k(y0, y1, src3, dst3)


def _dinv(deg_blk):
    return lax.rsqrt(deg_blk + 1.0)


def _tc_y0(x_pad, W, deg2):
    def body(xr, wr, dr, y0r, y1r):
        xw = jnp.dot(xr[...], wr[...], preferred_element_type=jnp.float32)
        y = xw * _dinv(dr[...])
        y0r[...] = y[:, :DH]
        y1r[...] = y[:, DH:]

    return pl.pallas_call(
        body,
        grid=(GRID,),
        in_specs=[
            pl.BlockSpec((BR, D), lambda i: (i, 0)),
            pl.BlockSpec((D, D), lambda i: (0, 0)),
            pl.BlockSpec((BR, 1), lambda i: (i, 0)),
        ],
        out_specs=[
            pl.BlockSpec((BR, DH), lambda i: (i, 0)),
            pl.BlockSpec((BR, DH), lambda i: (i, 0)),
        ],
        out_shape=[jax.ShapeDtypeStruct((N_PAD, DH), jnp.float32),
                   jax.ShapeDtypeStruct((N_PAD, DH), jnp.float32)],
    )(x_pad, W, deg2)


def _tc_bn_y(z, stats, g2, be2, W, deg2):
    def body(zr, str_, gr, ber, wr, dr, y0r, y1r):
        mu = str_[0:1, :] * (1.0 / N)
        var = str_[1:2, :] * (1.0 / N) - mu * mu
        scale = lax.rsqrt(var + EPS) * gr[...]
        h = jnp.maximum((zr[...] - mu) * scale + ber[...], 0.0)
        y = jnp.dot(h, wr[...], preferred_element_type=jnp.float32)
        y = y * _dinv(dr[...])
        y0r[...] = y[:, :DH]
        y1r[...] = y[:, DH:]

    return pl.pallas_call(
        body,
        grid=(GRID,),
        in_specs=[
            pl.BlockSpec((BR, D), lambda i: (i, 0)),
            pl.BlockSpec((2, D), lambda i: (0, 0)),
            pl.BlockSpec((1, D), lambda i: (0, 0)),
            pl.BlockSpec((1, D), lambda i: (0, 0)),
            pl.BlockSpec((D, D), lambda i: (0, 0)),
            pl.BlockSpec((BR, 1), lambda i: (i, 0)),
        ],
        out_specs=[
            pl.BlockSpec((BR, DH), lambda i: (i, 0)),
            pl.BlockSpec((BR, DH), lambda i: (i, 0)),
        ],
        out_shape=[jax.ShapeDtypeStruct((N_PAD, DH), jnp.float32),
                   jax.ShapeDtypeStruct((N_PAD, DH), jnp.float32)],
    )(z, stats, g2, be2, W, deg2)


def _tc_z_stats(S0, S1, deg2, b2):
    def body(s0r, s1r, dr, br_, zr, str_):
        i = pl.program_id(0)
        S = jnp.concatenate([s0r[...], s1r[...]], axis=1)
        z = S * _dinv(dr[...]) + br_[...]
        rows = lax.broadcasted_iota(jnp.int32, (BR, 1), 0) + i * BR
        z = jnp.where(rows < N, z, 0.0)
        zr[...] = z
        s1 = jnp.sum(z, axis=0, keepdims=True)
        s2 = jnp.sum(z * z, axis=0, keepdims=True)

        @pl.when(i == 0)
        def _():
            str_[...] = jnp.zeros_like(str_)

        str_[...] += jnp.concatenate([s1, s2], axis=0)

    return pl.pallas_call(
        body,
        grid=(GRID,),
        in_specs=[
            pl.BlockSpec((BR, DH), lambda i: (i, 0)),
            pl.BlockSpec((BR, DH), lambda i: (i, 0)),
            pl.BlockSpec((BR, 1), lambda i: (i, 0)),
            pl.BlockSpec((1, D), lambda i: (0, 0)),
        ],
        out_specs=[
            pl.BlockSpec((BR, D), lambda i: (i, 0)),
            pl.BlockSpec((2, D), lambda i: (0, 0)),
        ],
        out_shape=[jax.ShapeDtypeStruct((N_PAD, D), jnp.float32),
                   jax.ShapeDtypeStruct((2, D), jnp.float32)],
    )(S0, S1, deg2, b2)


def _tc_bn_out(z, stats, g2, be2):
    def body(zr, str_, gr, ber, outr):
        mu = str_[0:1, :] * (1.0 / N)
        var = str_[1:2, :] * (1.0 / N) - mu * mu
        scale = lax.rsqrt(var + EPS) * gr[...]
        outr[...] = jnp.maximum((zr[...] - mu) * scale + ber[...], 0.0)

    return pl.pallas_call(
        body,
        grid=(GRID,),
        in_specs=[
            pl.BlockSpec((BR, D), lambda i: (i, 0)),
            pl.BlockSpec((2, D), lambda i: (0, 0)),
            pl.BlockSpec((1, D), lambda i: (0, 0)),
            pl.BlockSpec((1, D), lambda i: (0, 0)),
        ],
        out_specs=pl.BlockSpec((BR, D), lambda i: (i, 0)),
        out_shape=jax.ShapeDtypeStruct((N_PAD, D), jnp.float32),
    )(z, stats, g2, be2)


def kernel(x, edge_index, W0, b0, g0, be0, W1, b1, g1, be1, W2, b2, g2, be2):
    E = edge_index.shape[1]
    n_chunks = -(-E // (NSUB * K))
    e_pad = NSUB * n_chunks * K
    pad_ar = jnp.arange(e_pad - E, dtype=jnp.int32)
    src = jnp.concatenate([edge_index[0], pad_ar % N]).reshape(
        NSUB, n_chunks, K)
    dst = jnp.concatenate([edge_index[1], N + pad_ar % (N_PAD - N)]).reshape(
        NSUB, n_chunks, K)
    x_pad = jnp.pad(x, ((0, N_PAD - N), (0, 0)))

    deg2 = _sc_degree(dst).reshape(N_PAD, 1)

    Ws = (W0, W1, W2)
    bs = (b0.reshape(1, D), b1.reshape(1, D), b2.reshape(1, D))
    gs = (g0.reshape(1, D), g1.reshape(1, D), g2.reshape(1, D))
    bes = (be0.reshape(1, D), be1.reshape(1, D), be2.reshape(1, D))

    z, stats = None, None
    for i in range(3):
        if i == 0:
            y0, y1 = _tc_y0(x_pad, Ws[i], deg2)
        else:
            y0, y1 = _tc_bn_y(z, stats, gs[i - 1], bes[i - 1], Ws[i], deg2)
        S0, S1 = _sc_scatter(y0, y1, src, dst)
        z, stats = _tc_z_stats(S0, S1, deg2, bs[i])
    out = _tc_bn_out(z, stats, gs[2], bes[2])
    return out[:N]


# speedup vs baseline: 1.0377x; 1.0377x over previous
import functools

import jax
import jax.numpy as jnp
from jax import lax
from jax.experimental import pallas as pl
from jax.experimental.pallas import tpu as pltpu
from jax.experimental.pallas import tpu_sc as plsc

N = 10000
D = 256
DH = 128
N_PAD = 10240
NSUB = 16
ROWS_PER_TILE = N_PAD // NSUB
K = 128
BR = 640
GRID = N_PAD // BR
EPS = 1e-5


def _sc_mesh():
    return plsc.VectorSubcoreMesh(core_axis_name="c", subcore_axis_name="s")


def _sc_degree(dst3):
    n_chunks = dst3.shape[1]

    @functools.partial(
        pl.kernel,
        out_type=jax.ShapeDtypeStruct((N_PAD,), jnp.float32),
        mesh=_sc_mesh(),
        scratch_types=[
            pltpu.VMEM((n_chunks, K), jnp.int32),
            pltpu.VMEM((K,), jnp.float32),
            pltpu.VMEM((ROWS_PER_TILE,), jnp.float32),
            pltpu.VMEM_SHARED((N_PAD,), jnp.float32),
        ],
    )
    def k(dst_hbm, deg_hbm, dst_v, ones_v, zrow_v, accum):
        c = lax.axis_index("c")
        s = lax.axis_index("s")

        @pl.when(c == 0)
        def _():
            row0 = s * ROWS_PER_TILE
            pltpu.sync_copy(dst_hbm.at[s], dst_v)
            for i in range(K // 16):
                ones_v[pl.ds(i * 16, 16)] = jnp.ones((16,), jnp.float32)
            for i in range(ROWS_PER_TILE // 16):
                zrow_v[pl.ds(i * 16, 16)] = jnp.zeros((16,), jnp.float32)
            pltpu.sync_copy(zrow_v, accum.at[pl.ds(row0, ROWS_PER_TILE)])
            plsc.subcore_barrier()

            def body(j, carry):
                pltpu.sync_copy(ones_v, accum.at[dst_v.at[j]], add=True)
                return carry

            lax.fori_loop(0, n_chunks, body, 0)
            plsc.subcore_barrier()
            pltpu.sync_copy(accum.at[pl.ds(row0, ROWS_PER_TILE)],
                            deg_hbm.at[pl.ds(row0, ROWS_PER_TILE)])

    return k(dst3)


def _sc_scatter(y0, y1, src3, dst3):
    n_chunks = src3.shape[1]
    ph0 = (n_chunks + 1) // 2
    phases = [(0, ph0), (ph0, n_chunks - ph0)]

    @functools.partial(
        pl.kernel,
        out_type=[jax.ShapeDtypeStruct((N_PAD, DH), jnp.float32),
                  jax.ShapeDtypeStruct((N_PAD, DH), jnp.float32)],
        mesh=_sc_mesh(),
        scratch_types=[
            pltpu.VMEM((ph0, K), jnp.int32),
            pltpu.VMEM((ph0, K), jnp.int32),
            pltpu.VMEM((K, DH), jnp.float32),
            pltpu.VMEM((K, DH), jnp.float32),
            pltpu.VMEM_SHARED((N_PAD, DH), jnp.float32),
            pltpu.SemaphoreType.DMA,
            pltpu.SemaphoreType.DMA,
            pltpu.SemaphoreType.DMA,
            pltpu.SemaphoreType.DMA,
        ],
    )
    def k(y0_hbm, y1_hbm, src_hbm, dst_hbm, s0_hbm, s1_hbm,
          src_v, dst_v, gbuf0, gbuf1, accum, sem0, sem1, ssem0, ssem1):
        c = lax.axis_index("c")
        s = lax.axis_index("s")
        row0 = s * ROWS_PER_TILE

        def half(y_hbm, s_out):
            def gather_start(j, buf, sem):
                pltpu.make_async_copy(
                    y_hbm.at[src_v.at[j, pl.ds(0, K // 2)]],
                    buf.at[pl.ds(0, K // 2)], sem).start()
                pltpu.make_async_copy(
                    y_hbm.at[src_v.at[j, pl.ds(K // 2, K // 2)]],
                    buf.at[pl.ds(K // 2, K // 2)], sem).start()

            def gather_wait(j, buf, sem):
                pltpu.make_async_copy(
                    y_hbm.at[src_v.at[j, pl.ds(0, K // 2)]],
                    buf.at[pl.ds(0, K // 2)], sem).wait()
                pltpu.make_async_copy(
                    y_hbm.at[src_v.at[j, pl.ds(K // 2, K // 2)]],
                    buf.at[pl.ds(K // 2, K // 2)], sem).wait()

            def scat(j, buf):
                pltpu.sync_copy(buf, accum.at[dst_v.at[j]], add=True)

            for pi, (lo, ph) in enumerate(phases):
                pltpu.sync_copy(src_hbm.at[s, pl.ds(lo, ph)],
                                src_v.at[pl.ds(0, ph)])
                pltpu.sync_copy(dst_hbm.at[s, pl.ds(lo, ph)],
                                dst_v.at[pl.ds(0, ph)])
                gather_start(0, gbuf0, sem0)
                gather_start(1, gbuf1, sem1)
                if pi == 0:
                    pltpu.sync_copy(y_hbm.at[pl.ds(row0, ROWS_PER_TILE)],
                                    accum.at[pl.ds(row0, ROWS_PER_TILE)])
                    plsc.subcore_barrier()

                def body(t, carry):
                    j = 2 * t
                    gather_wait(j, gbuf0, sem0)
                    scat(j, gbuf0)

                    @pl.when(j + 2 < ph)
                    def _():
                        gather_start(j + 2, gbuf0, sem0)

                    gather_wait(j + 1, gbuf1, sem1)
                    scat(j + 1, gbuf1)

                    @pl.when(j + 3 < ph)
                    def _():
                        gather_start(j + 3, gbuf1, sem1)

                    return carry

                lax.fori_loop(0, ph // 2, body, 0)
                if ph % 2:
                    gather_wait(ph - 1, gbuf0, sem0)
                    scat(ph - 1, gbuf0)

            plsc.subcore_barrier()
            pltpu.sync_copy(accum.at[pl.ds(row0, ROWS_PER_TILE)],
                            s_out.at[pl.ds(row0, ROWS_PER_TILE)])

        @pl.when(c == 0)
        def _():
            half(y0_hbm, s0_hbm)

        @pl.when(c == 1)
        def _():
            half(y1_hbm, s1_hbm)

    return k(y0, y1, src3, dst3)


def _dinv(deg_blk):
    return lax.rsqrt(deg_blk + 1.0)


def _tc_y0(x_pad, W, deg2):
    def body(xr, wr, dr, y0r, y1r):
        xw = jnp.dot(xr[...], wr[...], preferred_element_type=jnp.float32)
        y = xw * _dinv(dr[...])
        y0r[...] = y[:, :DH]
        y1r[...] = y[:, DH:]

    return pl.pallas_call(
        body,
        grid=(GRID,),
        in_specs=[
            pl.BlockSpec((BR, D), lambda i: (i, 0)),
            pl.BlockSpec((D, D), lambda i: (0, 0)),
            pl.BlockSpec((BR, 1), lambda i: (i, 0)),
        ],
        out_specs=[
            pl.BlockSpec((BR, DH), lambda i: (i, 0)),
            pl.BlockSpec((BR, DH), lambda i: (i, 0)),
        ],
        out_shape=[jax.ShapeDtypeStruct((N_PAD, DH), jnp.float32),
                   jax.ShapeDtypeStruct((N_PAD, DH), jnp.float32)],
    )(x_pad, W, deg2)


def _tc_bn_y(z, stats, g2, be2, W, deg2):
    def body(zr, str_, gr, ber, wr, dr, y0r, y1r):
        mu = str_[0:1, :] * (1.0 / N)
        var = str_[1:2, :] * (1.0 / N) - mu * mu
        scale = lax.rsqrt(var + EPS) * gr[...]
        h = jnp.maximum((zr[...] - mu) * scale + ber[...], 0.0)
        y = jnp.dot(h, wr[...], preferred_element_type=jnp.float32)
        y = y * _dinv(dr[...])
        y0r[...] = y[:, :DH]
        y1r[...] = y[:, DH:]

    return pl.pallas_call(
        body,
        grid=(GRID,),
        in_specs=[
            pl.BlockSpec((BR, D), lambda i: (i, 0)),
            pl.BlockSpec((2, D), lambda i: (0, 0)),
            pl.BlockSpec((1, D), lambda i: (0, 0)),
            pl.BlockSpec((1, D), lambda i: (0, 0)),
            pl.BlockSpec((D, D), lambda i: (0, 0)),
            pl.BlockSpec((BR, 1), lambda i: (i, 0)),
        ],
        out_specs=[
            pl.BlockSpec((BR, DH), lambda i: (i, 0)),
            pl.BlockSpec((BR, DH), lambda i: (i, 0)),
        ],
        out_shape=[jax.ShapeDtypeStruct((N_PAD, DH), jnp.float32),
                   jax.ShapeDtypeStruct((N_PAD, DH), jnp.float32)],
    )(z, stats, g2, be2, W, deg2)


def _tc_z_stats(S0, S1, deg2, b2):
    def body(s0r, s1r, dr, br_, zr, str_):
        i = pl.program_id(0)
        S = jnp.concatenate([s0r[...], s1r[...]], axis=1)
        z = S * _dinv(dr[...]) + br_[...]
        rows = lax.broadcasted_iota(jnp.int32, (BR, 1), 0) + i * BR
        z = jnp.where(rows < N, z, 0.0)
        zr[...] = z
        s1 = jnp.sum(z, axis=0, keepdims=True)
        s2 = jnp.sum(z * z, axis=0, keepdims=True)

        @pl.when(i == 0)
        def _():
            str_[...] = jnp.zeros_like(str_)

        str_[...] += jnp.concatenate([s1, s2], axis=0)

    return pl.pallas_call(
        body,
        grid=(GRID,),
        in_specs=[
            pl.BlockSpec((BR, DH), lambda i: (i, 0)),
            pl.BlockSpec((BR, DH), lambda i: (i, 0)),
            pl.BlockSpec((BR, 1), lambda i: (i, 0)),
            pl.BlockSpec((1, D), lambda i: (0, 0)),
        ],
        out_specs=[
            pl.BlockSpec((BR, D), lambda i: (i, 0)),
            pl.BlockSpec((2, D), lambda i: (0, 0)),
        ],
        out_shape=[jax.ShapeDtypeStruct((N_PAD, D), jnp.float32),
                   jax.ShapeDtypeStruct((2, D), jnp.float32)],
    )(S0, S1, deg2, b2)


def _tc_bn_out(z, stats, g2, be2):
    def body(zr, str_, gr, ber, outr):
        mu = str_[0:1, :] * (1.0 / N)
        var = str_[1:2, :] * (1.0 / N) - mu * mu
        scale = lax.rsqrt(var + EPS) * gr[...]
        outr[...] = jnp.maximum((zr[...] - mu) * scale + ber[...], 0.0)

    return pl.pallas_call(
        body,
        grid=(GRID,),
        in_specs=[
            pl.BlockSpec((BR, D), lambda i: (i, 0)),
            pl.BlockSpec((2, D), lambda i: (0, 0)),
            pl.BlockSpec((1, D), lambda i: (0, 0)),
            pl.BlockSpec((1, D), lambda i: (0, 0)),
        ],
        out_specs=pl.BlockSpec((BR, D), lambda i: (i, 0)),
        out_shape=jax.ShapeDtypeStruct((N_PAD, D), jnp.float32),
    )(z, stats, g2, be2)


def _zstats_phase(s0r, s1r, dr, br_, zs, sts, i):
    S = jnp.concatenate([s0r[...], s1r[...]], axis=1)
    z = S * _dinv(dr[...]) + br_[...]
    rows = lax.broadcasted_iota(jnp.int32, (BR, 1), 0) + i * BR
    z = jnp.where(rows < N, z, 0.0)
    zs[pl.ds(i * BR, BR), :] = z
    st = jnp.concatenate([jnp.sum(z, axis=0, keepdims=True),
                          jnp.sum(z * z, axis=0, keepdims=True)], axis=0)

    @pl.when(i == 0)
    def _():
        sts[...] = jnp.zeros_like(sts)

    sts[...] += st


def _bn_from_stats(sts, gr, ber, zblk):
    mu = sts[0:1, :] * (1.0 / N)
    var = sts[1:2, :] * (1.0 / N) - mu * mu
    scale = lax.rsqrt(var + EPS) * gr[...]
    return jnp.maximum((zblk - mu) * scale + ber[...], 0.0)


def _tc_fuse_mid(S0, S1, deg2, b2, g2, be2, W):
    def body(s0r, s1r, dr, br_, gr, ber, wr, y0r, y1r, zs, sts):
        p = pl.program_id(0)
        i = pl.program_id(1)

        @pl.when(p == 0)
        def _():
            _zstats_phase(s0r, s1r, dr, br_, zs, sts, i)

        @pl.when(p == 1)
        def _():
            h = _bn_from_stats(sts, gr, ber, zs[pl.ds(i * BR, BR), :])
            y = jnp.dot(h, wr[...], preferred_element_type=jnp.float32)
            y = y * _dinv(dr[...])
            y0r[...] = y[:, :DH]
            y1r[...] = y[:, DH:]

    return pl.pallas_call(
        body,
        grid=(2, GRID),
        in_specs=[
            pl.BlockSpec((BR, DH), lambda p, i: (i, 0)),
            pl.BlockSpec((BR, DH), lambda p, i: (i, 0)),
            pl.BlockSpec((BR, 1), lambda p, i: (i, 0)),
            pl.BlockSpec((1, D), lambda p, i: (0, 0)),
            pl.BlockSpec((1, D), lambda p, i: (0, 0)),
            pl.BlockSpec((1, D), lambda p, i: (0, 0)),
            pl.BlockSpec((D, D), lambda p, i: (0, 0)),
        ],
        out_specs=[
            pl.BlockSpec((BR, DH), lambda p, i: (i * p, 0)),
            pl.BlockSpec((BR, DH), lambda p, i: (i * p, 0)),
        ],
        out_shape=[jax.ShapeDtypeStruct((N_PAD, DH), jnp.float32),
                   jax.ShapeDtypeStruct((N_PAD, DH), jnp.float32)],
        scratch_shapes=[pltpu.VMEM((N_PAD, D), jnp.float32),
                        pltpu.VMEM((2, D), jnp.float32)],
    )(S0, S1, deg2, b2, g2, be2, W)


def _tc_fuse_last(S0, S1, deg2, b2, g2, be2):
    def body(s0r, s1r, dr, br_, gr, ber, outr, zs, sts):
        p = pl.program_id(0)
        i = pl.program_id(1)

        @pl.when(p == 0)
        def _():
            _zstats_phase(s0r, s1r, dr, br_, zs, sts, i)

        @pl.when(p == 1)
        def _():
            outr[...] = _bn_from_stats(sts, gr, ber, zs[pl.ds(i * BR, BR), :])

    return pl.pallas_call(
        body,
        grid=(2, GRID),
        in_specs=[
            pl.BlockSpec((BR, DH), lambda p, i: (i, 0)),
            pl.BlockSpec((BR, DH), lambda p, i: (i, 0)),
            pl.BlockSpec((BR, 1), lambda p, i: (i, 0)),
            pl.BlockSpec((1, D), lambda p, i: (0, 0)),
            pl.BlockSpec((1, D), lambda p, i: (0, 0)),
            pl.BlockSpec((1, D), lambda p, i: (0, 0)),
        ],
        out_specs=pl.BlockSpec((BR, D), lambda p, i: (i * p, 0)),
        out_shape=jax.ShapeDtypeStruct((N_PAD, D), jnp.float32),
        scratch_shapes=[pltpu.VMEM((N_PAD, D), jnp.float32),
                        pltpu.VMEM((2, D), jnp.float32)],
    )(S0, S1, deg2, b2, g2, be2)


def kernel(x, edge_index, W0, b0, g0, be0, W1, b1, g1, be1, W2, b2, g2, be2):
    E = edge_index.shape[1]
    n_chunks = -(-E // (NSUB * K))
    e_pad = NSUB * n_chunks * K
    pad_ar = jnp.arange(e_pad - E, dtype=jnp.int32)
    src = jnp.concatenate([edge_index[0], pad_ar % N]).reshape(
        NSUB, n_chunks, K)
    dst = jnp.concatenate([edge_index[1], N + pad_ar % (N_PAD - N)]).reshape(
        NSUB, n_chunks, K)
    x_pad = jnp.pad(x, ((0, N_PAD - N), (0, 0)))

    deg2 = _sc_degree(dst).reshape(N_PAD, 1)

    Ws = (W0, W1, W2)
    bs = (b0.reshape(1, D), b1.reshape(1, D), b2.reshape(1, D))
    gs = (g0.reshape(1, D), g1.reshape(1, D), g2.reshape(1, D))
    bes = (be0.reshape(1, D), be1.reshape(1, D), be2.reshape(1, D))

    y0, y1 = _tc_y0(x_pad, W0, deg2)
    for i in range(3):
        S0, S1 = _sc_scatter(y0, y1, src, dst)
        if i < 2:
            y0, y1 = _tc_fuse_mid(S0, S1, deg2, bs[i], gs[i], bes[i],
                                  Ws[i + 1])
        else:
            out = _tc_fuse_last(S0, S1, deg2, bs[2], gs[2], bes[2])
    return out[:N]
